# baseline (speedup 1.0000x reference)
import jax
import jax.numpy as jnp
from jax import lax
from jax.experimental import pallas as pl
from jax.experimental.pallas import tpu as pltpu
from jax.experimental.pallas import tpu_sc as plsc

_EPS = 1e-8
_NQ = 4096
_NKEY = 100000
_BLK = 128
_NB = 784
_KPAD = _NB * _BLK
_QT = 256
_KT = 2048
_TOPK = 50
_GCAP = 64
_CCAP = 96
_INT_MAX = 0x7FFFFFFF


def _dist_body(k_ref, q_ref, qsq_ref, ksq_ref, d_ref, mt_ref):
    kq = lax.dot_general(
        k_ref[...], q_ref[...], (((1,), (1,)), ((), ())),
        precision=lax.Precision.DEFAULT,
        preferred_element_type=jnp.float32)
    d2 = (qsq_ref[...] + ksq_ref[...]) - 2.0 * kq.T
    dist = jnp.sqrt(jnp.maximum(d2, 0.0) + _EPS)
    d_ref[...] = dist
    bmin = jnp.min(dist.reshape(_QT, _KT // _BLK, _BLK), axis=-1)
    mt_ref[...] = bmin.T


def _thresh_body(m_ref, t_ref):
    u = lax.bitcast_convert_type(m_ref[...], jnp.int32)
    lo0 = jnp.zeros((_QT, 1), jnp.int32)
    hi0 = jnp.full((_QT, 1), 0x7F800000, jnp.int32)

    def body(_, lh):
        lo, hi = lh
        mid = lo + lax.shift_right_logical(hi - lo, 1)
        cnt = jnp.sum((u <= mid).astype(jnp.int32), axis=1, keepdims=True)
        ge = cnt >= _TOPK
        return jnp.where(ge, lo, mid + 1), jnp.where(ge, mid, hi)

    _, hi = lax.fori_loop(0, 31, body, (lo0, hi0))
    t = lax.bitcast_convert_type(hi, jnp.float32)
    t_ref[...] = jnp.broadcast_to(t, (_QT, 16))


_NC, _NS, _NL = 2, 16, 16
_NW = _NC * _NS
_ROWS_PER = _NQ // _NW
_NVG = _NB // _NL


def _row_stage_a(r, minv, tq, gidx, lidx, rows, irows, sem_g,
                 drows_hbm, irows_hbm, lanes):
    tval = tq[...]
    pad_gid = r * _NB + (_NB - 1)
    for g in range(_GCAP // _NL):
        gidx[pl.ds(g * _NL, _NL)] = jnp.full((_NL,), pad_gid, jnp.int32)
        lidx[pl.ds(g * _NL, _NL)] = jnp.full((_NL,), _NB - 1, jnp.int32)

    def cand_body(j, cnt):
        m = minv[pl.ds(j * _NL, _NL)]
        mask = m <= tval
        cs = plsc.cumsum(mask.astype(jnp.int32))
        p = cnt + cs - 1
        ok = mask & (p < _GCAP)
        bid = j * _NL + lanes
        plsc.store_scatter(gidx, [p], r * _NB + bid, mask=ok)
        plsc.store_scatter(lidx, [p], bid, mask=ok)
        return cnt + plsc.all_reduce_population_count(mask)

    cnt = lax.fori_loop(0, _NVG, cand_body, jnp.zeros((_NL,), jnp.int32))
    nblk = jnp.minimum(lax.reduce_max(cnt, axes=(0,)), _GCAP)
    pltpu.async_copy(drows_hbm.at[gidx], rows, sem_g)
    pltpu.async_copy(irows_hbm.at[lidx], irows, sem_g)
    return nblk


def _row_stage_b(r, nblk, tq, gidx, lidx, rows, irows, sem_g,
                 cval, cidx, odbuf, oibuf, od_hbm, oi_hbm, lanes,
                 drows_hbm, irows_hbm):
    tval = tq[...]
    pltpu.make_async_copy(drows_hbm.at[pl.ds(0, _GCAP)], rows, sem_g).wait()
    pltpu.make_async_copy(irows_hbm.at[pl.ds(0, _GCAP)], irows, sem_g).wait()

    for g in range(_CCAP // _NL):
        cval[pl.ds(g * _NL, _NL)] = jnp.full((_NL,), jnp.inf, jnp.float32)
        cidx[pl.ds(g * _NL, _NL)] = jnp.full((_NL,), _INT_MAX, jnp.int32)

    def filt_body(b, c):
        for q in range(_BLK // _NL):
            v = rows.at[b][pl.ds(q * _NL, _NL)]
            ev = irows.at[b][pl.ds(q * _NL, _NL)]
            mask = v <= tval
            cs = plsc.cumsum(mask.astype(jnp.int32))
            p = c + cs - 1
            ok = mask & (p < _CCAP)
            plsc.store_scatter(cval, [p], v, mask=ok)
            plsc.store_scatter(cidx, [p], ev, mask=ok)
            c = c + plsc.all_reduce_population_count(mask)
        return c

    lax.fori_loop(0, nblk, filt_body, jnp.zeros((_NL,), jnp.int32))

    vs = [cval[pl.ds(g * _NL, _NL)] for g in range(_CCAP // _NL)]
    ks = [cidx[pl.ds(g * _NL, _NL)] for g in range(_CCAP // _NL)]

    def ext_body(t, carry):
        vs = carry
        m = vs[0]
        for g in range(1, _CCAP // _NL):
            m = jnp.minimum(m, vs[g])
        minval = lax.reduce_min(m, axes=(0,))
        eqs = [v == minval for v in vs]
        cand = jnp.where(eqs[0], ks[0], _INT_MAX)
        for g in range(1, _CCAP // _NL):
            cand = jnp.minimum(cand, jnp.where(eqs[g], ks[g], _INT_MAX))
        minidx = lax.reduce_min(cand, axes=(0,))
        tsplat = jnp.full((_NL,), t, jnp.int32)
        lane0 = lanes == 0
        plsc.store_scatter(odbuf, [tsplat],
                           jnp.full((_NL,), minval, jnp.float32), mask=lane0)
        plsc.store_scatter(oibuf, [tsplat],
                           jnp.full((_NL,), minidx, jnp.int32), mask=lane0)
        out = []
        for g in range(_CCAP // _NL):
            kill = eqs[g] & (ks[g] == minidx)
            out.append(jnp.where(kill, jnp.inf, vs[g]))
        return out

    lax.fori_loop(0, _TOPK, ext_body, vs)
    pltpu.sync_copy(odbuf, od_hbm.at[r])
    pltpu.sync_copy(oibuf, oi_hbm.at[r])


def _select_body(drows_hbm, irows_hbm, minima_hbm, t_hbm, od_hbm, oi_hbm,
                 minv0, minv1, tq0, tq1, gidx0, gidx1, lidx0, lidx1,
                 rows0, rows1, irows0, irows1, cval, cidx, odbuf, oibuf,
                 sem_m0, sem_m1, sem_g0, sem_g1):
    wid = lax.axis_index("s") * _NC + lax.axis_index("c")
    base = wid * _ROWS_PER
    lanes = lax.iota(jnp.int32, _NL)
    last = _NQ - 1

    def load_m(r, minv, tq, sem):
        rc = jnp.minimum(r, last)
        pltpu.async_copy(minima_hbm.at[rc], minv, sem)
        pltpu.async_copy(t_hbm.at[rc], tq, sem)

    def wait_m(minv, tq, sem):
        pltpu.make_async_copy(minima_hbm.at[0], minv, sem).wait()
        pltpu.make_async_copy(t_hbm.at[0], tq, sem).wait()

    s0 = (minv0, tq0, gidx0, lidx0, rows0, irows0, sem_m0, sem_g0)
    s1 = (minv1, tq1, gidx1, lidx1, rows1, irows1, sem_m1, sem_g1)

    def a_stage(r, s):
        minv, tq, gidx, lidx, rows, irows, sem_m, sem_g = s
        return _row_stage_a(r, minv, tq, gidx, lidx, rows, irows, sem_g,
                            drows_hbm, irows_hbm, lanes)

    def b_stage(r, nblk, s):
        minv, tq, gidx, lidx, rows, irows, sem_m, sem_g = s
        _row_stage_b(r, nblk, tq, gidx, lidx, rows, irows, sem_g,
                     cval, cidx, odbuf, oibuf, od_hbm, oi_hbm, lanes,
                     drows_hbm, irows_hbm)

    load_m(base, minv0, tq0, sem_m0)
    load_m(base + 1, minv1, tq1, sem_m1)
    wait_m(minv0, tq0, sem_m0)
    nblk0_init = a_stage(base, s0)

    def pair_body(rp, nblk0):
        r0 = base + 2 * rp
        wait_m(minv1, tq1, sem_m1)
        nblk1 = a_stage(r0 + 1, s1)
        load_m(r0 + 2, minv0, tq0, sem_m0)
        b_stage(r0, nblk0, s0)
        wait_m(minv0, tq0, sem_m0)
        nblk0n = a_stage(r0 + 2, s0)
        load_m(r0 + 3, minv1, tq1, sem_m1)
        b_stage(r0 + 1, nblk1, s1)
        return nblk0n

    lax.fori_loop(0, _ROWS_PER // 2, pair_body, nblk0_init)

    pltpu.make_async_copy(drows_hbm.at[pl.ds(0, _GCAP)], rows0, sem_g0).wait()
    pltpu.make_async_copy(irows_hbm.at[pl.ds(0, _GCAP)], irows0, sem_g0).wait()
    wait_m(minv1, tq1, sem_m1)


def _select(drows, irows, minima, trep):
    mesh = plsc.VectorSubcoreMesh(core_axis_name="c", subcore_axis_name="s")
    return pl.kernel(
        _select_body,
        out_type=[jax.ShapeDtypeStruct((_NQ, 64), jnp.float32),
                  jax.ShapeDtypeStruct((_NQ, 64), jnp.int32)],
        mesh=mesh,
        compiler_params=pltpu.CompilerParams(needs_layout_passes=False),
        scratch_types=[
            pltpu.VMEM((_NB,), jnp.float32),
            pltpu.VMEM((_NB,), jnp.float32),
            pltpu.VMEM((16,), jnp.float32),
            pltpu.VMEM((16,), jnp.float32),
            pltpu.VMEM((_GCAP,), jnp.int32),
            pltpu.VMEM((_GCAP,), jnp.int32),
            pltpu.VMEM((_GCAP,), jnp.int32),
            pltpu.VMEM((_GCAP,), jnp.int32),
            pltpu.VMEM((_GCAP, _BLK), jnp.float32),
            pltpu.VMEM((_GCAP, _BLK), jnp.float32),
            pltpu.VMEM((_GCAP, _BLK), jnp.int32),
            pltpu.VMEM((_GCAP, _BLK), jnp.int32),
            pltpu.VMEM((_CCAP,), jnp.float32),
            pltpu.VMEM((_CCAP,), jnp.int32),
            pltpu.VMEM((64,), jnp.float32),
            pltpu.VMEM((64,), jnp.int32),
            pltpu.SemaphoreType.DMA,
            pltpu.SemaphoreType.DMA,
            pltpu.SemaphoreType.DMA,
            pltpu.SemaphoreType.DMA,
        ],
    )(drows, irows, minima, trep)


def kernel(queries, keys):
    mean = jnp.mean(keys, axis=0)
    std = jnp.std(keys, axis=0)
    q = (queries - mean) / (std + _EPS)
    kk = (keys - mean) / (std + _EPS)
    q_sq = jnp.sum(q * q, axis=1, keepdims=True)
    k_sq = jnp.sum(kk * kk, axis=1)
    kkp = jnp.pad(kk, ((0, _KPAD - _NKEY), (0, 0)))
    ksq_p = jnp.pad(k_sq, (0, _KPAD - _NKEY),
                    constant_values=jnp.inf)[None, :]

    dists, minima_t = pl.pallas_call(
        _dist_body,
        grid=(_NQ // _QT, _KPAD // _KT),
        in_specs=[
            pl.BlockSpec((_KT, 128), lambda i, j: (j, 0)),
            pl.BlockSpec((_QT, 128), lambda i, j: (i, 0)),
            pl.BlockSpec((_QT, 1), lambda i, j: (i, 0)),
            pl.BlockSpec((1, _KT), lambda i, j: (0, j)),
        ],
        out_specs=[
            pl.BlockSpec((_QT, _KT), lambda i, j: (i, j)),
            pl.BlockSpec((_KT // _BLK, _QT), lambda i, j: (j, i)),
        ],
        out_shape=[jax.ShapeDtypeStruct((_NQ, _KPAD), jnp.float32),
                   jax.ShapeDtypeStruct((_NB, _NQ), jnp.float32)],
    )(kkp, q, q_sq, ksq_p)

    minima = minima_t.T

    thr = pl.pallas_call(
        _thresh_body,
        grid=(_NQ // _QT,),
        in_specs=[pl.BlockSpec((_QT, _NB), lambda i: (i, 0))],
        out_specs=pl.BlockSpec((_QT, 16), lambda i: (i, 0)),
        out_shape=jax.ShapeDtypeStruct((_NQ, 16), jnp.float32),
    )(minima)

    drows = dists.reshape(_NQ * _NB, _BLK)
    irows = (jnp.arange(_NB, dtype=jnp.int32)[:, None] * _BLK
             + jnp.arange(_BLK, dtype=jnp.int32)[None, :])
    od, oi = _select(drows, irows, minima, thr)
    return od[:, :_TOPK], oi[:, :_TOPK]


# speedup vs baseline: 1.0241x; 1.0241x over previous
import jax
import jax.numpy as jnp
from jax import lax
from jax.experimental import pallas as pl
from jax.experimental.pallas import tpu as pltpu
from jax.experimental.pallas import tpu_sc as plsc

_EPS = 1e-8
_NQ = 4096
_NKEY = 100000
_BLK = 128
_NB = 784
_KPAD = _NB * _BLK
_QT = 256
_KT = 2048
_TOPK = 50
_GCAP = 64
_CCAP = 96
_INT_MAX = 0x7FFFFFFF


def _trans_body(k_ref, kt_ref):
    kt_ref[...] = k_ref[...].T


def _dist_body(q_ref, kt_ref, qsq_ref, ksq_ref, d_ref, mt_ref):
    qk = lax.dot_general(
        q_ref[...], kt_ref[...], (((1,), (0,)), ((), ())),
        precision=lax.Precision.DEFAULT,
        preferred_element_type=jnp.float32)
    d2 = (qsq_ref[...] + ksq_ref[...]) - 2.0 * qk
    dist = jnp.sqrt(jnp.maximum(d2, 0.0) + _EPS)
    d_ref[...] = dist
    bmin = jnp.min(dist.reshape(_QT, _KT // _BLK, _BLK), axis=-1)
    mt_ref[...] = bmin.T


def _thresh_body(m_ref, t_ref):
    u = lax.bitcast_convert_type(m_ref[...], jnp.int32)
    lo0 = jnp.zeros((_QT, 1), jnp.int32)
    hi0 = jnp.full((_QT, 1), 0x7F800000, jnp.int32)

    def body(_, lh):
        lo, hi = lh
        mid = lo + lax.shift_right_logical(hi - lo, 1)
        cnt = jnp.sum((u <= mid).astype(jnp.int32), axis=1, keepdims=True)
        ge = cnt >= _TOPK
        return jnp.where(ge, lo, mid + 1), jnp.where(ge, mid, hi)

    _, hi = lax.fori_loop(0, 31, body, (lo0, hi0))
    t = lax.bitcast_convert_type(hi, jnp.float32)
    t_ref[...] = jnp.broadcast_to(t, (_QT, 16))


_NC, _NS, _NL = 2, 16, 16
_NW = _NC * _NS
_ROWS_PER = _NQ // _NW
_NVG = _NB // _NL


def _row_stage_a(r, minv, tq, gidx, lidx, rows, irows, sem_g,
                 drows_hbm, irows_hbm, lanes):
    tval = tq[...]
    pad_gid = r * _NB + (_NB - 1)
    for g in range(_GCAP // _NL):
        gidx[pl.ds(g * _NL, _NL)] = jnp.full((_NL,), pad_gid, jnp.int32)
        lidx[pl.ds(g * _NL, _NL)] = jnp.full((_NL,), _NB - 1, jnp.int32)

    def cand_body(j, cnt):
        m = minv[pl.ds(j * _NL, _NL)]
        mask = m <= tval
        cs = plsc.cumsum(mask.astype(jnp.int32))
        p = cnt + cs - 1
        ok = mask & (p < _GCAP)
        bid = j * _NL + lanes
        plsc.store_scatter(gidx, [p], r * _NB + bid, mask=ok)
        plsc.store_scatter(lidx, [p], bid, mask=ok)
        return cnt + plsc.all_reduce_population_count(mask)

    cnt = lax.fori_loop(0, _NVG, cand_body, jnp.zeros((_NL,), jnp.int32))
    nblk = jnp.minimum(lax.reduce_max(cnt, axes=(0,)), _GCAP)
    pltpu.async_copy(drows_hbm.at[gidx], rows, sem_g)
    pltpu.async_copy(irows_hbm.at[lidx], irows, sem_g)
    return nblk


def _row_stage_b(r, nblk, tq, gidx, lidx, rows, irows, sem_g,
                 cval, cidx, odbuf, oibuf, od_hbm, oi_hbm, lanes,
                 drows_hbm, irows_hbm):
    tval = tq[...]
    pltpu.make_async_copy(drows_hbm.at[pl.ds(0, _GCAP)], rows, sem_g).wait()
    pltpu.make_async_copy(irows_hbm.at[pl.ds(0, _GCAP)], irows, sem_g).wait()

    for g in range(_CCAP // _NL):
        cval[pl.ds(g * _NL, _NL)] = jnp.full((_NL,), jnp.inf, jnp.float32)
        cidx[pl.ds(g * _NL, _NL)] = jnp.full((_NL,), _INT_MAX, jnp.int32)

    def filt_body(b, c):
        for q in range(_BLK // _NL):
            v = rows.at[b][pl.ds(q * _NL, _NL)]
            mask = v <= tval
            pc = plsc.all_reduce_population_count(mask)

            def hit(c=c, v=v, mask=mask, pc=pc, b=b, q=q):
                ev = irows.at[b][pl.ds(q * _NL, _NL)]
                cs = plsc.cumsum(mask.astype(jnp.int32))
                p = c + cs - 1
                ok = mask & (p < _CCAP)
                plsc.store_scatter(cval, [p], v, mask=ok)
                plsc.store_scatter(cidx, [p], ev, mask=ok)
                return c + pc

            c = lax.cond(pc[0] > 0, hit, lambda c=c: c)
        return c

    lax.fori_loop(0, nblk, filt_body, jnp.zeros((_NL,), jnp.int32))

    vs = [cval[pl.ds(g * _NL, _NL)] for g in range(_CCAP // _NL)]
    ks = [cidx[pl.ds(g * _NL, _NL)] for g in range(_CCAP // _NL)]

    def ext_body(t, carry):
        vs = carry
        m = vs[0]
        for g in range(1, _CCAP // _NL):
            m = jnp.minimum(m, vs[g])
        minval = lax.reduce_min(m, axes=(0,))
        eqs = [v == minval for v in vs]
        cand = jnp.where(eqs[0], ks[0], _INT_MAX)
        for g in range(1, _CCAP // _NL):
            cand = jnp.minimum(cand, jnp.where(eqs[g], ks[g], _INT_MAX))
        minidx = lax.reduce_min(cand, axes=(0,))
        tsplat = jnp.full((_NL,), t, jnp.int32)
        lane0 = lanes == 0
        plsc.store_scatter(odbuf, [tsplat],
                           jnp.full((_NL,), minval, jnp.float32), mask=lane0)
        plsc.store_scatter(oibuf, [tsplat],
                           jnp.full((_NL,), minidx, jnp.int32), mask=lane0)
        out = []
        for g in range(_CCAP // _NL):
            kill = eqs[g] & (ks[g] == minidx)
            out.append(jnp.where(kill, jnp.inf, vs[g]))
        return out

    lax.fori_loop(0, _TOPK, ext_body, vs)
    pltpu.sync_copy(odbuf, od_hbm.at[r])
    pltpu.sync_copy(oibuf, oi_hbm.at[r])


def _select_body(drows_hbm, irows_hbm, minima_hbm, t_hbm, od_hbm, oi_hbm,
                 minv0, minv1, tq0, tq1, gidx0, gidx1, lidx0, lidx1,
                 rows0, rows1, irows0, irows1, cval, cidx, odbuf, oibuf,
                 sem_m0, sem_m1, sem_g0, sem_g1):
    wid = lax.axis_index("s") * _NC + lax.axis_index("c")
    base = wid * _ROWS_PER
    lanes = lax.iota(jnp.int32, _NL)
    last = _NQ - 1

    def load_m(r, minv, tq, sem):
        rc = jnp.minimum(r, last)
        pltpu.async_copy(minima_hbm.at[rc], minv, sem)
        pltpu.async_copy(t_hbm.at[rc], tq, sem)

    def wait_m(minv, tq, sem):
        pltpu.make_async_copy(minima_hbm.at[0], minv, sem).wait()
        pltpu.make_async_copy(t_hbm.at[0], tq, sem).wait()

    s0 = (minv0, tq0, gidx0, lidx0, rows0, irows0, sem_m0, sem_g0)
    s1 = (minv1, tq1, gidx1, lidx1, rows1, irows1, sem_m1, sem_g1)

    def a_stage(r, s):
        minv, tq, gidx, lidx, rows, irows, sem_m, sem_g = s
        return _row_stage_a(r, minv, tq, gidx, lidx, rows, irows, sem_g,
                            drows_hbm, irows_hbm, lanes)

    def b_stage(r, nblk, s):
        minv, tq, gidx, lidx, rows, irows, sem_m, sem_g = s
        _row_stage_b(r, nblk, tq, gidx, lidx, rows, irows, sem_g,
                     cval, cidx, odbuf, oibuf, od_hbm, oi_hbm, lanes,
                     drows_hbm, irows_hbm)

    load_m(base, minv0, tq0, sem_m0)
    load_m(base + 1, minv1, tq1, sem_m1)
    wait_m(minv0, tq0, sem_m0)
    nblk0_init = a_stage(base, s0)

    def pair_body(rp, nblk0):
        r0 = base + 2 * rp
        wait_m(minv1, tq1, sem_m1)
        nblk1 = a_stage(r0 + 1, s1)
        load_m(r0 + 2, minv0, tq0, sem_m0)
        b_stage(r0, nblk0, s0)
        wait_m(minv0, tq0, sem_m0)
        nblk0n = a_stage(r0 + 2, s0)
        load_m(r0 + 3, minv1, tq1, sem_m1)
        b_stage(r0 + 1, nblk1, s1)
        return nblk0n

    lax.fori_loop(0, _ROWS_PER // 2, pair_body, nblk0_init)

    pltpu.make_async_copy(drows_hbm.at[pl.ds(0, _GCAP)], rows0, sem_g0).wait()
    pltpu.make_async_copy(irows_hbm.at[pl.ds(0, _GCAP)], irows0, sem_g0).wait()
    wait_m(minv1, tq1, sem_m1)


def _select(drows, irows, minima, trep):
    mesh = plsc.VectorSubcoreMesh(core_axis_name="c", subcore_axis_name="s")
    return pl.kernel(
        _select_body,
        out_type=[jax.ShapeDtypeStruct((_NQ, 64), jnp.float32),
                  jax.ShapeDtypeStruct((_NQ, 64), jnp.int32)],
        mesh=mesh,
        compiler_params=pltpu.CompilerParams(needs_layout_passes=False),
        scratch_types=[
            pltpu.VMEM((_NB,), jnp.float32),
            pltpu.VMEM((_NB,), jnp.float32),
            pltpu.VMEM((16,), jnp.float32),
            pltpu.VMEM((16,), jnp.float32),
            pltpu.VMEM((_GCAP,), jnp.int32),
            pltpu.VMEM((_GCAP,), jnp.int32),
            pltpu.VMEM((_GCAP,), jnp.int32),
            pltpu.VMEM((_GCAP,), jnp.int32),
            pltpu.VMEM((_GCAP, _BLK), jnp.float32),
            pltpu.VMEM((_GCAP, _BLK), jnp.float32),
            pltpu.VMEM((_GCAP, _BLK), jnp.int32),
            pltpu.VMEM((_GCAP, _BLK), jnp.int32),
            pltpu.VMEM((_CCAP,), jnp.float32),
            pltpu.VMEM((_CCAP,), jnp.int32),
            pltpu.VMEM((64,), jnp.float32),
            pltpu.VMEM((64,), jnp.int32),
            pltpu.SemaphoreType.DMA,
            pltpu.SemaphoreType.DMA,
            pltpu.SemaphoreType.DMA,
            pltpu.SemaphoreType.DMA,
        ],
    )(drows, irows, minima, trep)


def kernel(queries, keys):
    mean = jnp.mean(keys, axis=0)
    std = jnp.std(keys, axis=0)
    q = (queries - mean) / (std + _EPS)
    kk = (keys - mean) / (std + _EPS)
    q_sq = jnp.sum(q * q, axis=1, keepdims=True)
    k_sq = jnp.sum(kk * kk, axis=1)
    kkp = jnp.pad(kk, ((0, _KPAD - _NKEY), (0, 0)))
    ksq_p = jnp.pad(k_sq, (0, _KPAD - _NKEY),
                    constant_values=jnp.inf)[None, :]

    kkt = pl.pallas_call(
        _trans_body,
        grid=(_KPAD // _KT,),
        in_specs=[pl.BlockSpec((_KT, 128), lambda j: (j, 0))],
        out_specs=pl.BlockSpec((128, _KT), lambda j: (0, j)),
        out_shape=jax.ShapeDtypeStruct((128, _KPAD), jnp.float32),
    )(kkp)

    dists, minima_t = pl.pallas_call(
        _dist_body,
        grid=(_NQ // _QT, _KPAD // _KT),
        in_specs=[
            pl.BlockSpec((_QT, 128), lambda i, j: (i, 0)),
            pl.BlockSpec((128, _KT), lambda i, j: (0, j)),
            pl.BlockSpec((_QT, 1), lambda i, j: (i, 0)),
            pl.BlockSpec((1, _KT), lambda i, j: (0, j)),
        ],
        out_specs=[
            pl.BlockSpec((_QT, _KT), lambda i, j: (i, j)),
            pl.BlockSpec((_KT // _BLK, _QT), lambda i, j: (j, i)),
        ],
        out_shape=[jax.ShapeDtypeStruct((_NQ, _KPAD), jnp.float32),
                   jax.ShapeDtypeStruct((_NB, _NQ), jnp.float32)],
    )(q, kkt, q_sq, ksq_p)

    minima = minima_t.T

    thr = pl.pallas_call(
        _thresh_body,
        grid=(_NQ // _QT,),
        in_specs=[pl.BlockSpec((_QT, _NB), lambda i: (i, 0))],
        out_specs=pl.BlockSpec((_QT, 16), lambda i: (i, 0)),
        out_shape=jax.ShapeDtypeStruct((_NQ, 16), jnp.float32),
    )(minima)

    drows = dists.reshape(_NQ * _NB, _BLK)
    irows = (jnp.arange(_NB, dtype=jnp.int32)[:, None] * _BLK
             + jnp.arange(_BLK, dtype=jnp.int32)[None, :])
    od, oi = _select(drows, irows, minima, thr)
    return od[:, :_TOPK], oi[:, :_TOPK]


# speedup vs baseline: 1.0544x; 1.0296x over previous
import jax
import jax.numpy as jnp
from jax import lax
from jax.experimental import pallas as pl
from jax.experimental.pallas import tpu as pltpu
from jax.experimental.pallas import tpu_sc as plsc

_EPS = 1e-8
_NQ = 4096
_NKEY = 100000
_BLK = 128
_NB = 784
_KPAD = _NB * _BLK
_QT = 256
_KT = 2048
_TOPK = 50
_GCAP = 64
_CCAP = 96
_INT_MAX = 0x7FFFFFFF


def _trans_body(k_ref, kt_ref):
    kt_ref[...] = k_ref[...].T


def _dist_body(q_ref, kt_ref, qsq_ref, ksq_ref, d_ref, mt_ref):
    qk = lax.dot_general(
        q_ref[...], kt_ref[...], (((1,), (0,)), ((), ())),
        precision=lax.Precision.DEFAULT,
        preferred_element_type=jnp.float32)
    d2 = (qsq_ref[...] + ksq_ref[...]) - 2.0 * qk
    dist = jnp.sqrt(jnp.maximum(d2, 0.0) + _EPS)
    d_ref[...] = dist
    bmin = jnp.min(dist.reshape(_QT, _KT // _BLK, _BLK), axis=-1)
    mt_ref[...] = bmin.T


def _thresh_body(m_ref, t_ref):
    u = lax.bitcast_convert_type(m_ref[...], jnp.int32)
    lo0 = jnp.zeros((_QT, 1), jnp.int32)
    hi0 = jnp.full((_QT, 1), 0x7F800000, jnp.int32)

    def body(_, lh):
        lo, hi = lh
        mid = lo + lax.shift_right_logical(hi - lo, 1)
        cnt = jnp.sum((u <= mid).astype(jnp.int32), axis=1, keepdims=True)
        ge = cnt >= _TOPK
        return jnp.where(ge, lo, mid + 1), jnp.where(ge, mid, hi)

    _, hi = lax.fori_loop(0, 31, body, (lo0, hi0))
    t = lax.bitcast_convert_type(hi, jnp.float32)
    t_ref[...] = jnp.broadcast_to(t, (_QT, 16))


_NC, _NS, _NL = 2, 16, 16
_NW = _NC * _NS
_CH = 2
_NQC = _NQ // _CH
_ROWS_PER = _NQC // _NW
_NVG = _NB // _NL


def _row_stage_a(r, minv, tq, gidx, lidx, rows, irows, sem_g,
                 drows_hbm, irows_hbm, lanes):
    tval = tq[...]
    pad_gid = r * _NB + (_NB - 1)
    for g in range(_GCAP // _NL):
        gidx[pl.ds(g * _NL, _NL)] = jnp.full((_NL,), pad_gid, jnp.int32)
        lidx[pl.ds(g * _NL, _NL)] = jnp.full((_NL,), _NB - 1, jnp.int32)

    def cand_body(j, cnt):
        m = minv[pl.ds(j * _NL, _NL)]
        mask = m <= tval
        cs = plsc.cumsum(mask.astype(jnp.int32))
        p = cnt + cs - 1
        ok = mask & (p < _GCAP)
        bid = j * _NL + lanes
        plsc.store_scatter(gidx, [p], r * _NB + bid, mask=ok)
        plsc.store_scatter(lidx, [p], bid, mask=ok)
        return cnt + plsc.all_reduce_population_count(mask)

    cnt = lax.fori_loop(0, _NVG, cand_body, jnp.zeros((_NL,), jnp.int32))
    nblk = jnp.minimum(lax.reduce_max(cnt, axes=(0,)), _GCAP)
    pltpu.async_copy(drows_hbm.at[gidx], rows, sem_g)
    pltpu.async_copy(irows_hbm.at[lidx], irows, sem_g)
    return nblk


def _row_stage_b(r, nblk, tq, gidx, lidx, rows, irows, sem_g,
                 cval, cidx, odbuf, oibuf, od_hbm, oi_hbm, lanes,
                 drows_hbm, irows_hbm):
    tval = tq[...]
    pltpu.make_async_copy(drows_hbm.at[pl.ds(0, _GCAP)], rows, sem_g).wait()
    pltpu.make_async_copy(irows_hbm.at[pl.ds(0, _GCAP)], irows, sem_g).wait()

    for g in range(_CCAP // _NL):
        cval[pl.ds(g * _NL, _NL)] = jnp.full((_NL,), jnp.inf, jnp.float32)
        cidx[pl.ds(g * _NL, _NL)] = jnp.full((_NL,), _INT_MAX, jnp.int32)

    def filt_body(b, c):
        for q in range(_BLK // _NL):
            v = rows.at[b][pl.ds(q * _NL, _NL)]
            mask = v <= tval
            pc = plsc.all_reduce_population_count(mask)

            def hit(c=c, v=v, mask=mask, pc=pc, b=b, q=q):
                ev = irows.at[b][pl.ds(q * _NL, _NL)]
                cs = plsc.cumsum(mask.astype(jnp.int32))
                p = c + cs - 1
                ok = mask & (p < _CCAP)
                plsc.store_scatter(cval, [p], v, mask=ok)
                plsc.store_scatter(cidx, [p], ev, mask=ok)
                return c + pc

            c = lax.cond(pc[0] > 0, hit, lambda c=c: c)
        return c

    lax.fori_loop(0, nblk, filt_body, jnp.zeros((_NL,), jnp.int32))

    vs = [cval[pl.ds(g * _NL, _NL)] for g in range(_CCAP // _NL)]
    ks = [cidx[pl.ds(g * _NL, _NL)] for g in range(_CCAP // _NL)]

    def ext_body(t, carry):
        vs = carry
        m = vs[0]
        for g in range(1, _CCAP // _NL):
            m = jnp.minimum(m, vs[g])
        minval = lax.reduce_min(m, axes=(0,))
        eqs = [v == minval for v in vs]
        cand = jnp.where(eqs[0], ks[0], _INT_MAX)
        for g in range(1, _CCAP // _NL):
            cand = jnp.minimum(cand, jnp.where(eqs[g], ks[g], _INT_MAX))
        minidx = lax.reduce_min(cand, axes=(0,))
        tsplat = jnp.full((_NL,), t, jnp.int32)
        lane0 = lanes == 0
        plsc.store_scatter(odbuf, [tsplat],
                           jnp.full((_NL,), minval, jnp.float32), mask=lane0)
        plsc.store_scatter(oibuf, [tsplat],
                           jnp.full((_NL,), minidx, jnp.int32), mask=lane0)
        out = []
        for g in range(_CCAP // _NL):
            kill = eqs[g] & (ks[g] == minidx)
            out.append(jnp.where(kill, jnp.inf, vs[g]))
        return out

    lax.fori_loop(0, _TOPK, ext_body, vs)
    pltpu.sync_copy(odbuf, od_hbm.at[r])
    pltpu.sync_copy(oibuf, oi_hbm.at[r])


def _select_body(drows_hbm, irows_hbm, minima_hbm, t_hbm, od_hbm, oi_hbm,
                 minv0, minv1, tq0, tq1, gidx0, gidx1, lidx0, lidx1,
                 rows0, rows1, irows0, irows1, cval, cidx, odbuf, oibuf,
                 sem_m0, sem_m1, sem_g0, sem_g1):
    wid = lax.axis_index("s") * _NC + lax.axis_index("c")
    base = wid * _ROWS_PER
    lanes = lax.iota(jnp.int32, _NL)
    last = _NQC - 1

    def load_m(r, minv, tq, sem):
        rc = jnp.minimum(r, last)
        pltpu.async_copy(minima_hbm.at[rc], minv, sem)
        pltpu.async_copy(t_hbm.at[rc], tq, sem)

    def wait_m(minv, tq, sem):
        pltpu.make_async_copy(minima_hbm.at[0], minv, sem).wait()
        pltpu.make_async_copy(t_hbm.at[0], tq, sem).wait()

    s0 = (minv0, tq0, gidx0, lidx0, rows0, irows0, sem_m0, sem_g0)
    s1 = (minv1, tq1, gidx1, lidx1, rows1, irows1, sem_m1, sem_g1)

    def a_stage(r, s):
        minv, tq, gidx, lidx, rows, irows, sem_m, sem_g = s
        return _row_stage_a(r, minv, tq, gidx, lidx, rows, irows, sem_g,
                            drows_hbm, irows_hbm, lanes)

    def b_stage(r, nblk, s):
        minv, tq, gidx, lidx, rows, irows, sem_m, sem_g = s
        _row_stage_b(r, nblk, tq, gidx, lidx, rows, irows, sem_g,
                     cval, cidx, odbuf, oibuf, od_hbm, oi_hbm, lanes,
                     drows_hbm, irows_hbm)

    load_m(base, minv0, tq0, sem_m0)
    load_m(base + 1, minv1, tq1, sem_m1)
    wait_m(minv0, tq0, sem_m0)
    nblk0_init = a_stage(base, s0)

    def pair_body(rp, nblk0):
        r0 = base + 2 * rp
        wait_m(minv1, tq1, sem_m1)
        nblk1 = a_stage(r0 + 1, s1)
        load_m(r0 + 2, minv0, tq0, sem_m0)
        b_stage(r0, nblk0, s0)
        wait_m(minv0, tq0, sem_m0)
        nblk0n = a_stage(r0 + 2, s0)
        load_m(r0 + 3, minv1, tq1, sem_m1)
        b_stage(r0 + 1, nblk1, s1)
        return nblk0n

    lax.fori_loop(0, _ROWS_PER // 2, pair_body, nblk0_init)

    pltpu.make_async_copy(drows_hbm.at[pl.ds(0, _GCAP)], rows0, sem_g0).wait()
    pltpu.make_async_copy(irows_hbm.at[pl.ds(0, _GCAP)], irows0, sem_g0).wait()
    wait_m(minv1, tq1, sem_m1)


def _select(drows, irows, minima, trep):
    mesh = plsc.VectorSubcoreMesh(core_axis_name="c", subcore_axis_name="s")
    return pl.kernel(
        _select_body,
        out_type=[jax.ShapeDtypeStruct((_NQC, 64), jnp.float32),
                  jax.ShapeDtypeStruct((_NQC, 64), jnp.int32)],
        mesh=mesh,
        compiler_params=pltpu.CompilerParams(needs_layout_passes=False),
        scratch_types=[
            pltpu.VMEM((_NB,), jnp.float32),
            pltpu.VMEM((_NB,), jnp.float32),
            pltpu.VMEM((16,), jnp.float32),
            pltpu.VMEM((16,), jnp.float32),
            pltpu.VMEM((_GCAP,), jnp.int32),
            pltpu.VMEM((_GCAP,), jnp.int32),
            pltpu.VMEM((_GCAP,), jnp.int32),
            pltpu.VMEM((_GCAP,), jnp.int32),
            pltpu.VMEM((_GCAP, _BLK), jnp.float32),
            pltpu.VMEM((_GCAP, _BLK), jnp.float32),
            pltpu.VMEM((_GCAP, _BLK), jnp.int32),
            pltpu.VMEM((_GCAP, _BLK), jnp.int32),
            pltpu.VMEM((_CCAP,), jnp.float32),
            pltpu.VMEM((_CCAP,), jnp.int32),
            pltpu.VMEM((64,), jnp.float32),
            pltpu.VMEM((64,), jnp.int32),
            pltpu.SemaphoreType.DMA,
            pltpu.SemaphoreType.DMA,
            pltpu.SemaphoreType.DMA,
            pltpu.SemaphoreType.DMA,
        ],
    )(drows, irows, minima, trep)


def kernel(queries, keys):
    mean = jnp.mean(keys, axis=0)
    std = jnp.std(keys, axis=0)
    q = (queries - mean) / (std + _EPS)
    kk = (keys - mean) / (std + _EPS)
    q_sq = jnp.sum(q * q, axis=1, keepdims=True)
    k_sq = jnp.sum(kk * kk, axis=1)
    kkp = jnp.pad(kk, ((0, _KPAD - _NKEY), (0, 0)))
    ksq_p = jnp.pad(k_sq, (0, _KPAD - _NKEY),
                    constant_values=jnp.inf)[None, :]

    kkt = pl.pallas_call(
        _trans_body,
        grid=(_KPAD // _KT,),
        in_specs=[pl.BlockSpec((_KT, 128), lambda j: (j, 0))],
        out_specs=pl.BlockSpec((128, _KT), lambda j: (0, j)),
        out_shape=jax.ShapeDtypeStruct((128, _KPAD), jnp.float32),
    )(kkp)

    irows = (jnp.arange(_NB, dtype=jnp.int32)[:, None] * _BLK
             + jnp.arange(_BLK, dtype=jnp.int32)[None, :])

    ods, ois = [], []
    for ci in range(_CH):
        qoff = ci * (_NQC // _QT)
        dists, minima_t = pl.pallas_call(
            _dist_body,
            grid=(_NQC // _QT, _KPAD // _KT),
            in_specs=[
                pl.BlockSpec((_QT, 128), lambda i, j, qoff=qoff: (i + qoff, 0)),
                pl.BlockSpec((128, _KT), lambda i, j: (0, j)),
                pl.BlockSpec((_QT, 1), lambda i, j, qoff=qoff: (i + qoff, 0)),
                pl.BlockSpec((1, _KT), lambda i, j: (0, j)),
            ],
            out_specs=[
                pl.BlockSpec((_QT, _KT), lambda i, j: (i, j)),
                pl.BlockSpec((_KT // _BLK, _QT), lambda i, j: (j, i)),
            ],
            out_shape=[jax.ShapeDtypeStruct((_NQC, _KPAD), jnp.float32),
                       jax.ShapeDtypeStruct((_NB, _NQC), jnp.float32)],
        )(q, kkt, q_sq, ksq_p)

        minima = minima_t.T

        thr = pl.pallas_call(
            _thresh_body,
            grid=(_NQC // _QT,),
            in_specs=[pl.BlockSpec((_QT, _NB), lambda i: (i, 0))],
            out_specs=pl.BlockSpec((_QT, 16), lambda i: (i, 0)),
            out_shape=jax.ShapeDtypeStruct((_NQC, 16), jnp.float32),
        )(minima)

        drows = dists.reshape(_NQC * _NB, _BLK)
        od, oi = _select(drows, irows, minima, thr)
        ods.append(od)
        ois.append(oi)

    od = jnp.concatenate(ods, axis=0)
    oi = jnp.concatenate(ois, axis=0)
    return od[:, :_TOPK], oi[:, :_TOPK]


# speedup vs baseline: 1.0653x; 1.0103x over previous
import jax
import jax.numpy as jnp
from jax import lax
from jax.experimental import pallas as pl
from jax.experimental.pallas import tpu as pltpu
from jax.experimental.pallas import tpu_sc as plsc

_EPS = 1e-8
_NQ = 4096
_NKEY = 100000
_BLK = 128
_NB = 784
_KPAD = _NB * _BLK
_QT = 256
_KT = 2048
_TOPK = 50
_GCAP = 64
_CCAP = 96
_INT_MAX = 0x7FFFFFFF


def _trans_body(k_ref, kt_ref):
    kt_ref[...] = k_ref[...].T


def _dist_body(q_ref, kt_ref, qsq_ref, ksq_ref, d_ref, mt_ref):
    qk = lax.dot_general(
        q_ref[...], kt_ref[...], (((1,), (0,)), ((), ())),
        precision=lax.Precision.DEFAULT,
        preferred_element_type=jnp.float32)
    d2 = (qsq_ref[...] + ksq_ref[...]) - 2.0 * qk
    dist = jnp.sqrt(jnp.maximum(d2, 0.0) + _EPS)
    d_ref[...] = dist
    bmin = jnp.min(dist.reshape(_QT, _KT // _BLK, _BLK), axis=-1)
    mt_ref[...] = bmin.T


def _thresh_body(m_ref, t_ref):
    u = lax.bitcast_convert_type(m_ref[...], jnp.int32)
    lo0 = jnp.zeros((_QT, 1), jnp.int32)
    hi0 = jnp.full((_QT, 1), 0x7F800000, jnp.int32)

    def body(_, lh):
        lo, hi = lh
        mid = lo + lax.shift_right_logical(hi - lo, 1)
        cnt = jnp.sum((u <= mid).astype(jnp.int32), axis=1, keepdims=True)
        ge = cnt >= _TOPK
        return jnp.where(ge, lo, mid + 1), jnp.where(ge, mid, hi)

    _, hi = lax.fori_loop(0, 31, body, (lo0, hi0))
    t = lax.bitcast_convert_type(hi, jnp.float32)
    t_ref[...] = jnp.broadcast_to(t, (_QT, 16))


_NC, _NS, _NL = 2, 16, 16
_NW = _NC * _NS
_CH = 4
_NQC = _NQ // _CH
_ROWS_PER = _NQC // _NW
_NVG = _NB // _NL


def _row_stage_a(r, minv, tq, gidx, lidx, rows, irows, sem_g,
                 drows_hbm, irows_hbm, lanes):
    tval = tq[...]
    pad_gid = r * _NB + (_NB - 1)
    for g in range(_GCAP // _NL):
        gidx[pl.ds(g * _NL, _NL)] = jnp.full((_NL,), pad_gid, jnp.int32)
        lidx[pl.ds(g * _NL, _NL)] = jnp.full((_NL,), _NB - 1, jnp.int32)

    def cand_body(j, cnt):
        m = minv[pl.ds(j * _NL, _NL)]
        mask = m <= tval
        cs = plsc.cumsum(mask.astype(jnp.int32))
        p = cnt + cs - 1
        ok = mask & (p < _GCAP)
        bid = j * _NL + lanes
        plsc.store_scatter(gidx, [p], r * _NB + bid, mask=ok)
        plsc.store_scatter(lidx, [p], bid, mask=ok)
        return cnt + plsc.all_reduce_population_count(mask)

    cnt = lax.fori_loop(0, _NVG, cand_body, jnp.zeros((_NL,), jnp.int32))
    nblk = jnp.minimum(lax.reduce_max(cnt, axes=(0,)), _GCAP)
    pltpu.async_copy(drows_hbm.at[gidx], rows, sem_g)
    pltpu.async_copy(irows_hbm.at[lidx], irows, sem_g)
    return nblk


def _row_stage_b(r, nblk, tq, gidx, lidx, rows, irows, sem_g,
                 cval, cidx, odbuf, oibuf, od_hbm, oi_hbm, lanes,
                 drows_hbm, irows_hbm):
    tval = tq[...]
    pltpu.make_async_copy(drows_hbm.at[pl.ds(0, _GCAP)], rows, sem_g).wait()
    pltpu.make_async_copy(irows_hbm.at[pl.ds(0, _GCAP)], irows, sem_g).wait()

    for g in range(_CCAP // _NL):
        cval[pl.ds(g * _NL, _NL)] = jnp.full((_NL,), jnp.inf, jnp.float32)
        cidx[pl.ds(g * _NL, _NL)] = jnp.full((_NL,), _INT_MAX, jnp.int32)

    def filt_body(b, c):
        for q in range(_BLK // _NL):
            v = rows.at[b][pl.ds(q * _NL, _NL)]
            mask = v <= tval
            pc = plsc.all_reduce_population_count(mask)

            def hit(c=c, v=v, mask=mask, pc=pc, b=b, q=q):
                ev = irows.at[b][pl.ds(q * _NL, _NL)]
                cs = plsc.cumsum(mask.astype(jnp.int32))
                p = c + cs - 1
                ok = mask & (p < _CCAP)
                plsc.store_scatter(cval, [p], v, mask=ok)
                plsc.store_scatter(cidx, [p], ev, mask=ok)
                return c + pc

            c = lax.cond(pc[0] > 0, hit, lambda c=c: c)
        return c

    lax.fori_loop(0, nblk, filt_body, jnp.zeros((_NL,), jnp.int32))

    vs = [cval[pl.ds(g * _NL, _NL)] for g in range(_CCAP // _NL)]
    ks = [cidx[pl.ds(g * _NL, _NL)] for g in range(_CCAP // _NL)]

    def ext_body(t, carry):
        vs = carry
        m = vs[0]
        for g in range(1, _CCAP // _NL):
            m = jnp.minimum(m, vs[g])
        minval = lax.reduce_min(m, axes=(0,))
        eqs = [v == minval for v in vs]
        cand = jnp.where(eqs[0], ks[0], _INT_MAX)
        for g in range(1, _CCAP // _NL):
            cand = jnp.minimum(cand, jnp.where(eqs[g], ks[g], _INT_MAX))
        minidx = lax.reduce_min(cand, axes=(0,))
        tsplat = jnp.full((_NL,), t, jnp.int32)
        lane0 = lanes == 0
        plsc.store_scatter(odbuf, [tsplat],
                           jnp.full((_NL,), minval, jnp.float32), mask=lane0)
        plsc.store_scatter(oibuf, [tsplat],
                           jnp.full((_NL,), minidx, jnp.int32), mask=lane0)
        out = []
        for g in range(_CCAP // _NL):
            kill = eqs[g] & (ks[g] == minidx)
            out.append(jnp.where(kill, jnp.inf, vs[g]))
        return out

    lax.fori_loop(0, _TOPK, ext_body, vs)
    pltpu.sync_copy(odbuf, od_hbm.at[r])
    pltpu.sync_copy(oibuf, oi_hbm.at[r])


def _select_body(drows_hbm, irows_hbm, minima_hbm, t_hbm, od_hbm, oi_hbm,
                 minv0, minv1, tq0, tq1, gidx0, gidx1, lidx0, lidx1,
                 rows0, rows1, irows0, irows1, cval, cidx, odbuf, oibuf,
                 sem_m0, sem_m1, sem_g0, sem_g1):
    wid = lax.axis_index("s") * _NC + lax.axis_index("c")
    base = wid * _ROWS_PER
    lanes = lax.iota(jnp.int32, _NL)
    last = _NQC - 1

    def load_m(r, minv, tq, sem):
        rc = jnp.minimum(r, last)
        pltpu.async_copy(minima_hbm.at[rc], minv, sem)
        pltpu.async_copy(t_hbm.at[rc], tq, sem)

    def wait_m(minv, tq, sem):
        pltpu.make_async_copy(minima_hbm.at[0], minv, sem).wait()
        pltpu.make_async_copy(t_hbm.at[0], tq, sem).wait()

    s0 = (minv0, tq0, gidx0, lidx0, rows0, irows0, sem_m0, sem_g0)
    s1 = (minv1, tq1, gidx1, lidx1, rows1, irows1, sem_m1, sem_g1)

    def a_stage(r, s):
        minv, tq, gidx, lidx, rows, irows, sem_m, sem_g = s
        return _row_stage_a(r, minv, tq, gidx, lidx, rows, irows, sem_g,
                            drows_hbm, irows_hbm, lanes)

    def b_stage(r, nblk, s):
        minv, tq, gidx, lidx, rows, irows, sem_m, sem_g = s
        _row_stage_b(r, nblk, tq, gidx, lidx, rows, irows, sem_g,
                     cval, cidx, odbuf, oibuf, od_hbm, oi_hbm, lanes,
                     drows_hbm, irows_hbm)

    load_m(base, minv0, tq0, sem_m0)
    load_m(base + 1, minv1, tq1, sem_m1)
    wait_m(minv0, tq0, sem_m0)
    nblk0_init = a_stage(base, s0)

    def pair_body(rp, nblk0):
        r0 = base + 2 * rp
        wait_m(minv1, tq1, sem_m1)
        nblk1 = a_stage(r0 + 1, s1)
        load_m(r0 + 2, minv0, tq0, sem_m0)
        b_stage(r0, nblk0, s0)
        wait_m(minv0, tq0, sem_m0)
        nblk0n = a_stage(r0 + 2, s0)
        load_m(r0 + 3, minv1, tq1, sem_m1)
        b_stage(r0 + 1, nblk1, s1)
        return nblk0n

    lax.fori_loop(0, _ROWS_PER // 2, pair_body, nblk0_init)

    pltpu.make_async_copy(drows_hbm.at[pl.ds(0, _GCAP)], rows0, sem_g0).wait()
    pltpu.make_async_copy(irows_hbm.at[pl.ds(0, _GCAP)], irows0, sem_g0).wait()
    wait_m(minv1, tq1, sem_m1)


def _select(drows, irows, minima, trep):
    mesh = plsc.VectorSubcoreMesh(core_axis_name="c", subcore_axis_name="s")
    return pl.kernel(
        _select_body,
        out_type=[jax.ShapeDtypeStruct((_NQC, 64), jnp.float32),
                  jax.ShapeDtypeStruct((_NQC, 64), jnp.int32)],
        mesh=mesh,
        compiler_params=pltpu.CompilerParams(needs_layout_passes=False),
        scratch_types=[
            pltpu.VMEM((_NB,), jnp.float32),
            pltpu.VMEM((_NB,), jnp.float32),
            pltpu.VMEM((16,), jnp.float32),
            pltpu.VMEM((16,), jnp.float32),
            pltpu.VMEM((_GCAP,), jnp.int32),
            pltpu.VMEM((_GCAP,), jnp.int32),
            pltpu.VMEM((_GCAP,), jnp.int32),
            pltpu.VMEM((_GCAP,), jnp.int32),
            pltpu.VMEM((_GCAP, _BLK), jnp.float32),
            pltpu.VMEM((_GCAP, _BLK), jnp.float32),
            pltpu.VMEM((_GCAP, _BLK), jnp.int32),
            pltpu.VMEM((_GCAP, _BLK), jnp.int32),
            pltpu.VMEM((_CCAP,), jnp.float32),
            pltpu.VMEM((_CCAP,), jnp.int32),
            pltpu.VMEM((64,), jnp.float32),
            pltpu.VMEM((64,), jnp.int32),
            pltpu.SemaphoreType.DMA,
            pltpu.SemaphoreType.DMA,
            pltpu.SemaphoreType.DMA,
            pltpu.SemaphoreType.DMA,
        ],
    )(drows, irows, minima, trep)


def kernel(queries, keys):
    mean = jnp.mean(keys, axis=0)
    std = jnp.std(keys, axis=0)
    q = (queries - mean) / (std + _EPS)
    kk = (keys - mean) / (std + _EPS)
    q_sq = jnp.sum(q * q, axis=1, keepdims=True)
    k_sq = jnp.sum(kk * kk, axis=1)
    kkp = jnp.pad(kk, ((0, _KPAD - _NKEY), (0, 0)))
    ksq_p = jnp.pad(k_sq, (0, _KPAD - _NKEY),
                    constant_values=jnp.inf)[None, :]

    kkt = pl.pallas_call(
        _trans_body,
        grid=(_KPAD // _KT,),
        in_specs=[pl.BlockSpec((_KT, 128), lambda j: (j, 0))],
        out_specs=pl.BlockSpec((128, _KT), lambda j: (0, j)),
        out_shape=jax.ShapeDtypeStruct((128, _KPAD), jnp.float32),
    )(kkp)

    irows = (jnp.arange(_NB, dtype=jnp.int32)[:, None] * _BLK
             + jnp.arange(_BLK, dtype=jnp.int32)[None, :])

    ods, ois = [], []
    for ci in range(_CH):
        qoff = ci * (_NQC // _QT)
        dists, minima_t = pl.pallas_call(
            _dist_body,
            grid=(_NQC // _QT, _KPAD // _KT),
            in_specs=[
                pl.BlockSpec((_QT, 128), lambda i, j, qoff=qoff: (i + qoff, 0)),
                pl.BlockSpec((128, _KT), lambda i, j: (0, j)),
                pl.BlockSpec((_QT, 1), lambda i, j, qoff=qoff: (i + qoff, 0)),
                pl.BlockSpec((1, _KT), lambda i, j: (0, j)),
            ],
            out_specs=[
                pl.BlockSpec((_QT, _KT), lambda i, j: (i, j)),
                pl.BlockSpec((_KT // _BLK, _QT), lambda i, j: (j, i)),
            ],
            out_shape=[jax.ShapeDtypeStruct((_NQC, _KPAD), jnp.float32),
                       jax.ShapeDtypeStruct((_NB, _NQC), jnp.float32)],
        )(q, kkt, q_sq, ksq_p)

        minima = minima_t.T

        thr = pl.pallas_call(
            _thresh_body,
            grid=(_NQC // _QT,),
            in_specs=[pl.BlockSpec((_QT, _NB), lambda i: (i, 0))],
            out_specs=pl.BlockSpec((_QT, 16), lambda i: (i, 0)),
            out_shape=jax.ShapeDtypeStruct((_NQC, 16), jnp.float32),
        )(minima)

        drows = dists.reshape(_NQC * _NB, _BLK)
        od, oi = _select(drows, irows, minima, thr)
        ods.append(od)
        ois.append(oi)

    od = jnp.concatenate(ods, axis=0)
    oi = jnp.concatenate(ois, axis=0)
    return od[:, :_TOPK], oi[:, :_TOPK]
